# baseline (speedup 1.0000x reference)
import jax
import jax.numpy as jnp
from jax import lax
from jax.experimental import pallas as pl
from jax.experimental.pallas import tpu as pltpu
from jax.experimental.pallas import tpu_sc as plsc

N = 10000
D = 256
E = 160000

NC = 2
NS = 16
NW = NC * NS
LANES = 16
DD = 2 * D
SLOTS = 6
CH = 32
NCH = 168
EPW = NCH * CH

ROWS_TC = 1000


def _pack2(lo_f32, hi_f32):
    lo = lax.bitcast_convert_type(lo_f32.astype(jnp.bfloat16), jnp.uint16)
    hi = lax.bitcast_convert_type(hi_f32.astype(jnp.bfloat16), jnp.uint16)
    word = lo.astype(jnp.uint32) | (hi.astype(jnp.uint32) << 16)
    return lax.bitcast_convert_type(word, jnp.int32)


def _tc_pack_body(zs_ref, zo_ref, zi_ref, wi_ref, bi_ref, wo_ref, bo_ref,
                  p_ref, q_ref):
    zs = zs_ref[...]
    a_in = lax.dot_general(zs, wi_ref[...], (((1,), (1,)), ((), ())),
                           preferred_element_type=jnp.float32) + bi_ref[...]
    a_out = lax.dot_general(zs, wo_ref[...], (((1,), (1,)), ((), ())),
                            preferred_element_type=jnp.float32) + bo_ref[...]
    p_ref[...] = _pack2(zo_ref[...], a_in)
    q_ref[...] = _pack2(a_out, zi_ref[...])


def _build_pq(z_in, z_out, z_self, W_in, b_in, W_out, b_out):
    b_in2 = b_in.reshape(1, D)
    b_out2 = b_out.reshape(1, D)
    grid = (N // ROWS_TC,)
    row_spec = pl.BlockSpec((ROWS_TC, D), lambda i: (i, 0))
    full_spec = pl.BlockSpec((D, D), lambda i: (0, 0))
    bias_spec = pl.BlockSpec((1, D), lambda i: (0, 0))
    out_spec = pl.BlockSpec((ROWS_TC, D), lambda i: (i, 0))
    return pl.pallas_call(
        _tc_pack_body,
        grid=grid,
        in_specs=[row_spec, row_spec, row_spec, full_spec, bias_spec,
                  full_spec, bias_spec],
        out_specs=[out_spec, out_spec],
        out_shape=[jax.ShapeDtypeStruct((N, D), jnp.int32),
                   jax.ShapeDtypeStruct((N, D), jnp.int32)],
    )(z_self, z_out, z_in, W_in, b_in2, W_out, b_out2)


def _sc_edge_body(p_hbm, q_hbm, ei_hbm, out_hbm,
                  src_all, dst_all, out_all, p_rows, q_rows, accbuf,
                  *sems):
    wid = lax.axis_index("s") * NC + lax.axis_index("c")
    base = jnp.minimum(wid * EPW, E - EPW)
    pltpu.sync_copy(ei_hbm.at[0, pl.ds(base, EPW)], src_all)
    pltpu.sync_copy(ei_hbm.at[1, pl.ds(base, EPW)], dst_all)
    lane16 = lax.iota(jnp.int32, LANES) * LANES

    def issue(c, b):
        pltpu.async_copy(p_hbm.at[src_all.at[pl.ds(c * CH, CH)]],
                         p_rows.at[b], sems[2 * b])
        pltpu.async_copy(q_hbm.at[dst_all.at[pl.ds(c * CH, CH)]],
                         q_rows.at[b], sems[2 * b + 1])

    def drain(c, b):
        pltpu.make_async_copy(p_hbm.at[src_all.at[pl.ds(c * CH, CH)]],
                              p_rows.at[b], sems[2 * b]).wait()
        pltpu.make_async_copy(q_hbm.at[dst_all.at[pl.ds(c * CH, CH)]],
                              q_rows.at[b], sems[2 * b + 1]).wait()

    def compute(c, b):
        for sub in range(CH // LANES):
            @plsc.parallel_loop(0, LANES, step=1, unroll=2)
            def edge(e):
                r = sub * LANES + e
                acc0 = jnp.zeros((LANES,), jnp.float32)
                acc1 = jnp.zeros((LANES,), jnp.float32)
                for j in range(DD // (2 * LANES)):
                    pw = p_rows[b, r, pl.ds(j * LANES, LANES)]
                    qw = q_rows[b, r, pl.ds(j * LANES, LANES)]
                    prod = (plsc.bitcast(pw, jnp.bfloat16)
                            * plsc.bitcast(qw, jnp.bfloat16))
                    pa, pb = plsc.unpack(prod,
                                         format=plsc.PackFormat.INTERLEAVED)
                    if j % 2 == 0:
                        acc0 = acc0 + (pa + pb)
                    else:
                        acc1 = acc1 + (pa + pb)
                accbuf[pl.ds(e * LANES, LANES)] = acc0 + acc1
            totals = plsc.load_gather(accbuf, [lane16])
            for j in range(1, LANES):
                totals = totals + plsc.load_gather(accbuf, [lane16 + j])
            out_all[pl.ds(c * CH + sub * LANES, LANES)] = (
                1.0 / (1.0 + jnp.exp(-0.5 * totals)))

    for b in range(SLOTS - 1):
        issue(b, b)

    def step(k, carry):
        for b in range(SLOTS):
            c = SLOTS * k + b
            drain(c, b)
            issue(jnp.minimum(c + SLOTS - 1, NCH - 1), (b + SLOTS - 1) % SLOTS)
            compute(c, b)
        return carry

    lax.fori_loop(0, NCH // SLOTS, step, 0)
    for b in range(SLOTS - 1):
        drain(NCH - 1, b)
    pltpu.sync_copy(out_all, out_hbm.at[pl.ds(base, EPW)])


def _edge_scores(P, Q, edge_index):
    mesh = plsc.VectorSubcoreMesh(core_axis_name="c", subcore_axis_name="s",
                                  num_cores=NC, num_subcores=NS)
    run = pl.kernel(
        _sc_edge_body,
        out_type=jax.ShapeDtypeStruct((E,), jnp.float32),
        mesh=mesh,
        compiler_params=pltpu.CompilerParams(needs_layout_passes=False),
        scratch_types=[
            pltpu.VMEM((EPW,), jnp.int32),
            pltpu.VMEM((EPW,), jnp.int32),
            pltpu.VMEM((EPW,), jnp.float32),
            pltpu.VMEM((SLOTS, CH, DD // 2), jnp.int32),
            pltpu.VMEM((SLOTS, CH, DD // 2), jnp.int32),
            pltpu.VMEM((LANES * LANES,), jnp.float32),
        ] + [pltpu.SemaphoreType.DMA] * (2 * SLOTS),
    )
    return run(P, Q, edge_index)


def kernel(z_in, z_out, z_self, edge_index, W_in, b_in, W_out, b_out):
    P32, Q32 = _build_pq(z_in, z_out, z_self, W_in, b_in, W_out, b_out)
    return _edge_scores(P32, Q32, edge_index.astype(jnp.int32))


# speedup vs baseline: 1.0004x; 1.0004x over previous
import jax
import jax.numpy as jnp
from jax import lax
from jax.experimental import pallas as pl
from jax.experimental.pallas import tpu as pltpu
from jax.experimental.pallas import tpu_sc as plsc

N = 10000
D = 256
E = 160000

NC = 2
NS = 16
NW = NC * NS
LANES = 16
DD = 2 * D
SLOTS = 3
CH = 64
NCH = 84
EPW = NCH * CH

ROWS_TC = 1000


def _pack2(lo_f32, hi_f32):
    lo = lax.bitcast_convert_type(lo_f32.astype(jnp.bfloat16), jnp.uint16)
    hi = lax.bitcast_convert_type(hi_f32.astype(jnp.bfloat16), jnp.uint16)
    word = lo.astype(jnp.uint32) | (hi.astype(jnp.uint32) << 16)
    return lax.bitcast_convert_type(word, jnp.int32)


def _tc_pack_body(zs_ref, zo_ref, zi_ref, wi_ref, bi_ref, wo_ref, bo_ref,
                  p_ref, q_ref):
    zs = zs_ref[...]
    a_in = lax.dot_general(zs, wi_ref[...], (((1,), (1,)), ((), ())),
                           preferred_element_type=jnp.float32) + bi_ref[...]
    a_out = lax.dot_general(zs, wo_ref[...], (((1,), (1,)), ((), ())),
                            preferred_element_type=jnp.float32) + bo_ref[...]
    p_ref[...] = _pack2(zo_ref[...], a_in)
    q_ref[...] = _pack2(a_out, zi_ref[...])


def _build_pq(z_in, z_out, z_self, W_in, b_in, W_out, b_out):
    b_in2 = b_in.reshape(1, D)
    b_out2 = b_out.reshape(1, D)
    grid = (N // ROWS_TC,)
    row_spec = pl.BlockSpec((ROWS_TC, D), lambda i: (i, 0))
    full_spec = pl.BlockSpec((D, D), lambda i: (0, 0))
    bias_spec = pl.BlockSpec((1, D), lambda i: (0, 0))
    out_spec = pl.BlockSpec((ROWS_TC, D), lambda i: (i, 0))
    return pl.pallas_call(
        _tc_pack_body,
        grid=grid,
        in_specs=[row_spec, row_spec, row_spec, full_spec, bias_spec,
                  full_spec, bias_spec],
        out_specs=[out_spec, out_spec],
        out_shape=[jax.ShapeDtypeStruct((N, D), jnp.int32),
                   jax.ShapeDtypeStruct((N, D), jnp.int32)],
    )(z_self, z_out, z_in, W_in, b_in2, W_out, b_out2)


def _sc_edge_body(p_hbm, q_hbm, ei_hbm, out_hbm,
                  src_all, dst_all, out_all, p_rows, q_rows, accbuf,
                  *sems):
    wid = lax.axis_index("s") * NC + lax.axis_index("c")
    base = jnp.minimum(wid * EPW, E - EPW)
    pltpu.sync_copy(ei_hbm.at[0, pl.ds(base, EPW)], src_all)
    pltpu.sync_copy(ei_hbm.at[1, pl.ds(base, EPW)], dst_all)
    lane16 = lax.iota(jnp.int32, LANES) * LANES

    def issue(c, b):
        pltpu.async_copy(p_hbm.at[src_all.at[pl.ds(c * CH, CH)]],
                         p_rows.at[b], sems[2 * b])
        pltpu.async_copy(q_hbm.at[dst_all.at[pl.ds(c * CH, CH)]],
                         q_rows.at[b], sems[2 * b + 1])

    def drain(c, b):
        pltpu.make_async_copy(p_hbm.at[src_all.at[pl.ds(c * CH, CH)]],
                              p_rows.at[b], sems[2 * b]).wait()
        pltpu.make_async_copy(q_hbm.at[dst_all.at[pl.ds(c * CH, CH)]],
                              q_rows.at[b], sems[2 * b + 1]).wait()

    def compute(c, b):
        for sub in range(CH // LANES):
            @plsc.parallel_loop(0, LANES, step=1, unroll=2)
            def edge(e):
                r = sub * LANES + e
                acc0 = jnp.zeros((LANES,), jnp.float32)
                acc1 = jnp.zeros((LANES,), jnp.float32)
                for j in range(DD // (2 * LANES)):
                    pw = p_rows[b, r, pl.ds(j * LANES, LANES)]
                    qw = q_rows[b, r, pl.ds(j * LANES, LANES)]
                    prod = (plsc.bitcast(pw, jnp.bfloat16)
                            * plsc.bitcast(qw, jnp.bfloat16))
                    pa, pb = plsc.unpack(prod,
                                         format=plsc.PackFormat.INTERLEAVED)
                    if j % 2 == 0:
                        acc0 = acc0 + (pa + pb)
                    else:
                        acc1 = acc1 + (pa + pb)
                accbuf[pl.ds(e * LANES, LANES)] = acc0 + acc1
            totals = plsc.load_gather(accbuf, [lane16])
            for j in range(1, LANES):
                totals = totals + plsc.load_gather(accbuf, [lane16 + j])
            out_all[pl.ds(c * CH + sub * LANES, LANES)] = (
                1.0 / (1.0 + jnp.exp(-0.5 * totals)))

    for b in range(SLOTS - 1):
        issue(b, b)

    def step(k, carry):
        for b in range(SLOTS):
            c = SLOTS * k + b
            drain(c, b)
            issue(jnp.minimum(c + SLOTS - 1, NCH - 1), (b + SLOTS - 1) % SLOTS)
            compute(c, b)
        return carry

    lax.fori_loop(0, NCH // SLOTS, step, 0)
    for b in range(SLOTS - 1):
        drain(NCH - 1, b)
    pltpu.sync_copy(out_all, out_hbm.at[pl.ds(base, EPW)])


def _edge_scores(P, Q, edge_index):
    mesh = plsc.VectorSubcoreMesh(core_axis_name="c", subcore_axis_name="s",
                                  num_cores=NC, num_subcores=NS)
    run = pl.kernel(
        _sc_edge_body,
        out_type=jax.ShapeDtypeStruct((E,), jnp.float32),
        mesh=mesh,
        compiler_params=pltpu.CompilerParams(needs_layout_passes=False),
        scratch_types=[
            pltpu.VMEM((EPW,), jnp.int32),
            pltpu.VMEM((EPW,), jnp.int32),
            pltpu.VMEM((EPW,), jnp.float32),
            pltpu.VMEM((SLOTS, CH, DD // 2), jnp.int32),
            pltpu.VMEM((SLOTS, CH, DD // 2), jnp.int32),
            pltpu.VMEM((LANES * LANES,), jnp.float32),
        ] + [pltpu.SemaphoreType.DMA] * (2 * SLOTS),
    )
    return run(P, Q, edge_index)


def kernel(z_in, z_out, z_self, edge_index, W_in, b_in, W_out, b_out):
    P32, Q32 = _build_pq(z_in, z_out, z_self, W_in, b_in, W_out, b_out)
    return _edge_scores(P32, Q32, edge_index.astype(jnp.int32))


# speedup vs baseline: 1.0469x; 1.0465x over previous
import jax
import jax.numpy as jnp
from jax import lax
from jax.experimental import pallas as pl
from jax.experimental.pallas import tpu as pltpu
from jax.experimental.pallas import tpu_sc as plsc

N = 10000
D = 256
E = 160000

NC = 2
NS = 16
NW = NC * NS
LANES = 16
DD = 2 * D
SLOTS = 5
CH = 32
NCH = 160
EPW = NCH * CH

ROWS_TC = 1000


def _pack2(lo_f32, hi_f32):
    lo = lax.bitcast_convert_type(lo_f32.astype(jnp.bfloat16), jnp.uint16)
    hi = lax.bitcast_convert_type(hi_f32.astype(jnp.bfloat16), jnp.uint16)
    word = lo.astype(jnp.uint32) | (hi.astype(jnp.uint32) << 16)
    return lax.bitcast_convert_type(word, jnp.int32)


def _tc_pack_body(zs_ref, zo_ref, zi_ref, wi_ref, bi_ref, wo_ref, bo_ref,
                  p_ref, q_ref):
    zs = zs_ref[...]
    a_in = lax.dot_general(zs, wi_ref[...], (((1,), (1,)), ((), ())),
                           preferred_element_type=jnp.float32) + bi_ref[...]
    a_out = lax.dot_general(zs, wo_ref[...], (((1,), (1,)), ((), ())),
                            preferred_element_type=jnp.float32) + bo_ref[...]
    p_ref[...] = _pack2(zo_ref[...], a_in)
    q_ref[...] = _pack2(a_out, zi_ref[...])


def _build_pq(z_in, z_out, z_self, W_in, b_in, W_out, b_out):
    b_in2 = b_in.reshape(1, D)
    b_out2 = b_out.reshape(1, D)
    grid = (N // ROWS_TC,)
    row_spec = pl.BlockSpec((ROWS_TC, D), lambda i: (i, 0))
    full_spec = pl.BlockSpec((D, D), lambda i: (0, 0))
    bias_spec = pl.BlockSpec((1, D), lambda i: (0, 0))
    out_spec = pl.BlockSpec((ROWS_TC, D), lambda i: (i, 0))
    return pl.pallas_call(
        _tc_pack_body,
        grid=grid,
        in_specs=[row_spec, row_spec, row_spec, full_spec, bias_spec,
                  full_spec, bias_spec],
        out_specs=[out_spec, out_spec],
        out_shape=[jax.ShapeDtypeStruct((N, D), jnp.int32),
                   jax.ShapeDtypeStruct((N, D), jnp.int32)],
    )(z_self, z_out, z_in, W_in, b_in2, W_out, b_out2)


def _sc_edge_body(p_hbm, q_hbm, ei_hbm, out_hbm,
                  src_all, dst_all, out_all, p_rows, q_rows, accbuf,
                  *sems):
    wid = lax.axis_index("s") * NC + lax.axis_index("c")
    base = jnp.minimum(wid * EPW, E - EPW)
    pltpu.sync_copy(ei_hbm.at[0, pl.ds(base, EPW)], src_all)
    pltpu.sync_copy(ei_hbm.at[1, pl.ds(base, EPW)], dst_all)
    lane16 = lax.iota(jnp.int32, LANES) * LANES

    def issue(c, b):
        pltpu.async_copy(p_hbm.at[src_all.at[pl.ds(c * CH, CH)]],
                         p_rows.at[b], sems[2 * b])
        pltpu.async_copy(q_hbm.at[dst_all.at[pl.ds(c * CH, CH)]],
                         q_rows.at[b], sems[2 * b + 1])

    def drain(c, b):
        pltpu.make_async_copy(p_hbm.at[src_all.at[pl.ds(c * CH, CH)]],
                              p_rows.at[b], sems[2 * b]).wait()
        pltpu.make_async_copy(q_hbm.at[dst_all.at[pl.ds(c * CH, CH)]],
                              q_rows.at[b], sems[2 * b + 1]).wait()

    def compute(c, b):
        for sub in range(CH // LANES):
            @plsc.parallel_loop(0, LANES, step=1, unroll=2)
            def edge(e):
                r = sub * LANES + e
                acc0 = jnp.zeros((LANES,), jnp.float32)
                acc1 = jnp.zeros((LANES,), jnp.float32)
                for j in range(DD // (2 * LANES)):
                    pw = p_rows[b, r, pl.ds(j * LANES, LANES)]
                    qw = q_rows[b, r, pl.ds(j * LANES, LANES)]
                    prod = (plsc.bitcast(pw, jnp.bfloat16)
                            * plsc.bitcast(qw, jnp.bfloat16))
                    pa, pb = plsc.unpack(prod,
                                         format=plsc.PackFormat.INTERLEAVED)
                    if j % 2 == 0:
                        acc0 = acc0 + (pa + pb)
                    else:
                        acc1 = acc1 + (pa + pb)
                accbuf[pl.ds(e * LANES, LANES)] = acc0 + acc1
            totals = plsc.load_gather(accbuf, [lane16])
            for j in range(1, LANES):
                totals = totals + plsc.load_gather(accbuf, [lane16 + j])
            out_all[pl.ds(c * CH + sub * LANES, LANES)] = (
                1.0 / (1.0 + jnp.exp(-0.5 * totals)))

    for b in range(SLOTS - 1):
        issue(b, b)

    def step(k, carry):
        for b in range(SLOTS):
            c = SLOTS * k + b
            drain(c, b)
            issue(jnp.minimum(c + SLOTS - 1, NCH - 1), (b + SLOTS - 1) % SLOTS)
            compute(c, b)
        return carry

    lax.fori_loop(0, NCH // SLOTS, step, 0)
    for b in range(SLOTS - 1):
        drain(NCH - 1, b)
    pltpu.sync_copy(out_all, out_hbm.at[pl.ds(base, EPW)])


def _edge_scores(P, Q, edge_index):
    mesh = plsc.VectorSubcoreMesh(core_axis_name="c", subcore_axis_name="s",
                                  num_cores=NC, num_subcores=NS)
    run = pl.kernel(
        _sc_edge_body,
        out_type=jax.ShapeDtypeStruct((E,), jnp.float32),
        mesh=mesh,
        compiler_params=pltpu.CompilerParams(needs_layout_passes=False),
        scratch_types=[
            pltpu.VMEM((EPW,), jnp.int32),
            pltpu.VMEM((EPW,), jnp.int32),
            pltpu.VMEM((EPW,), jnp.float32),
            pltpu.VMEM((SLOTS, CH, DD // 2), jnp.int32),
            pltpu.VMEM((SLOTS, CH, DD // 2), jnp.int32),
            pltpu.VMEM((LANES * LANES,), jnp.float32),
        ] + [pltpu.SemaphoreType.DMA] * (2 * SLOTS),
    )
    return run(P, Q, edge_index)


def kernel(z_in, z_out, z_self, edge_index, W_in, b_in, W_out, b_out):
    P32, Q32 = _build_pq(z_in, z_out, z_self, W_in, b_in, W_out, b_out)
    return _edge_scores(P32, Q32, edge_index.astype(jnp.int32))


# speedup vs baseline: 1.0833x; 1.0348x over previous
import jax
import jax.numpy as jnp
from jax import lax
from jax.experimental import pallas as pl
from jax.experimental.pallas import tpu as pltpu
from jax.experimental.pallas import tpu_sc as plsc

N = 10000
D = 256
E = 160000

NC = 2
NS = 16
NW = NC * NS
LANES = 16
DD = 2 * D
SLOTS = 4
CH = 32
NCH = 160
EPW = NCH * CH

ROWS_TC = 1000


def _pack2(lo_f32, hi_f32):
    lo = lax.bitcast_convert_type(lo_f32.astype(jnp.bfloat16), jnp.uint16)
    hi = lax.bitcast_convert_type(hi_f32.astype(jnp.bfloat16), jnp.uint16)
    word = lo.astype(jnp.uint32) | (hi.astype(jnp.uint32) << 16)
    return lax.bitcast_convert_type(word, jnp.int32)


def _tc_pack_body(zs_ref, zo_ref, zi_ref, wi_ref, bi_ref, wo_ref, bo_ref,
                  p_ref, q_ref):
    zs = zs_ref[...]
    a_in = lax.dot_general(zs, wi_ref[...], (((1,), (1,)), ((), ())),
                           preferred_element_type=jnp.float32) + bi_ref[...]
    a_out = lax.dot_general(zs, wo_ref[...], (((1,), (1,)), ((), ())),
                            preferred_element_type=jnp.float32) + bo_ref[...]
    p_ref[...] = _pack2(zo_ref[...], a_in)
    q_ref[...] = _pack2(a_out, zi_ref[...])


def _build_pq(z_in, z_out, z_self, W_in, b_in, W_out, b_out):
    b_in2 = b_in.reshape(1, D)
    b_out2 = b_out.reshape(1, D)
    grid = (N // ROWS_TC,)
    row_spec = pl.BlockSpec((ROWS_TC, D), lambda i: (i, 0))
    full_spec = pl.BlockSpec((D, D), lambda i: (0, 0))
    bias_spec = pl.BlockSpec((1, D), lambda i: (0, 0))
    out_spec = pl.BlockSpec((ROWS_TC, D), lambda i: (i, 0))
    return pl.pallas_call(
        _tc_pack_body,
        grid=grid,
        in_specs=[row_spec, row_spec, row_spec, full_spec, bias_spec,
                  full_spec, bias_spec],
        out_specs=[out_spec, out_spec],
        out_shape=[jax.ShapeDtypeStruct((N, D), jnp.int32),
                   jax.ShapeDtypeStruct((N, D), jnp.int32)],
    )(z_self, z_out, z_in, W_in, b_in2, W_out, b_out2)


def _sc_edge_body(p_hbm, q_hbm, ei_hbm, out_hbm,
                  src_all, dst_all, out_all, p_rows, q_rows, accbuf,
                  *sems):
    wid = lax.axis_index("s") * NC + lax.axis_index("c")
    base = jnp.minimum(wid * EPW, E - EPW)
    pltpu.sync_copy(ei_hbm.at[0, pl.ds(base, EPW)], src_all)
    pltpu.sync_copy(ei_hbm.at[1, pl.ds(base, EPW)], dst_all)
    lane16 = lax.iota(jnp.int32, LANES) * LANES

    def issue(c, b):
        pltpu.async_copy(p_hbm.at[src_all.at[pl.ds(c * CH, CH)]],
                         p_rows.at[b], sems[2 * b])
        pltpu.async_copy(q_hbm.at[dst_all.at[pl.ds(c * CH, CH)]],
                         q_rows.at[b], sems[2 * b + 1])

    def drain(c, b):
        pltpu.make_async_copy(p_hbm.at[src_all.at[pl.ds(c * CH, CH)]],
                              p_rows.at[b], sems[2 * b]).wait()
        pltpu.make_async_copy(q_hbm.at[dst_all.at[pl.ds(c * CH, CH)]],
                              q_rows.at[b], sems[2 * b + 1]).wait()

    def compute(c, b):
        for sub in range(CH // LANES):
            @plsc.parallel_loop(0, LANES, step=1, unroll=2)
            def edge(e):
                r = sub * LANES + e
                acc0 = jnp.zeros((LANES,), jnp.float32)
                acc1 = jnp.zeros((LANES,), jnp.float32)
                for j in range(DD // (2 * LANES)):
                    pw = p_rows[b, r, pl.ds(j * LANES, LANES)]
                    qw = q_rows[b, r, pl.ds(j * LANES, LANES)]
                    prod = (plsc.bitcast(pw, jnp.bfloat16)
                            * plsc.bitcast(qw, jnp.bfloat16))
                    pa, pb = plsc.unpack(prod,
                                         format=plsc.PackFormat.INTERLEAVED)
                    if j % 2 == 0:
                        acc0 = acc0 + (pa + pb)
                    else:
                        acc1 = acc1 + (pa + pb)
                accbuf[pl.ds(e * LANES, LANES)] = acc0 + acc1
            totals = plsc.load_gather(accbuf, [lane16])
            for j in range(1, LANES):
                totals = totals + plsc.load_gather(accbuf, [lane16 + j])
            out_all[pl.ds(c * CH + sub * LANES, LANES)] = (
                1.0 / (1.0 + jnp.exp(-0.5 * totals)))

    for b in range(SLOTS - 1):
        issue(b, b)

    def step(k, carry):
        for b in range(SLOTS):
            c = SLOTS * k + b
            drain(c, b)
            issue(jnp.minimum(c + SLOTS - 1, NCH - 1), (b + SLOTS - 1) % SLOTS)
            compute(c, b)
        return carry

    lax.fori_loop(0, NCH // SLOTS, step, 0)
    for b in range(SLOTS - 1):
        drain(NCH - 1, b)
    pltpu.sync_copy(out_all, out_hbm.at[pl.ds(base, EPW)])


def _edge_scores(P, Q, edge_index):
    mesh = plsc.VectorSubcoreMesh(core_axis_name="c", subcore_axis_name="s",
                                  num_cores=NC, num_subcores=NS)
    run = pl.kernel(
        _sc_edge_body,
        out_type=jax.ShapeDtypeStruct((E,), jnp.float32),
        mesh=mesh,
        compiler_params=pltpu.CompilerParams(needs_layout_passes=False),
        scratch_types=[
            pltpu.VMEM((EPW,), jnp.int32),
            pltpu.VMEM((EPW,), jnp.int32),
            pltpu.VMEM((EPW,), jnp.float32),
            pltpu.VMEM((SLOTS, CH, DD // 2), jnp.int32),
            pltpu.VMEM((SLOTS, CH, DD // 2), jnp.int32),
            pltpu.VMEM((LANES * LANES,), jnp.float32),
        ] + [pltpu.SemaphoreType.DMA] * (2 * SLOTS),
    )
    return run(P, Q, edge_index)


def kernel(z_in, z_out, z_self, edge_index, W_in, b_in, W_out, b_out):
    P32, Q32 = _build_pq(z_in, z_out, z_self, W_in, b_in, W_out, b_out)
    return _edge_scores(P32, Q32, edge_index.astype(jnp.int32))
